# baseline (speedup 1.0000x reference)
import functools

import jax
import jax.numpy as jnp
from jax import lax
from jax.experimental import pallas as pl
from jax.experimental.pallas import tpu as pltpu
from jax.experimental.pallas import tpu_sc as plsc

D = 64
BATCH = 4096
HIST = 200
B = BATCH * HIST
NC, NS = 2, 16
NW = NC * NS
XROWS_W = BATCH // NW
B_PER_W = XROWS_W * HIST
GR = 4
GROUP = GR * HIST
NG = XROWS_W // GR

_mesh = plsc.VectorSubcoreMesh(core_axis_name="c", subcore_axis_name="s")


@functools.partial(
    pl.kernel,
    mesh=_mesh,
    out_type=jax.ShapeDtypeStruct((B, D), jnp.float32),
    compiler_params=pltpu.CompilerParams(use_tc_tiling_on_sc=False),
    scratch_types=[
        pltpu.VMEM((XROWS_W, HIST), jnp.int32),
        pltpu.VMEM((GROUP, D), jnp.float32),
        pltpu.VMEM((GROUP, D), jnp.float32),
        pltpu.SemaphoreType.DMA,
        pltpu.SemaphoreType.DMA,
        pltpu.SemaphoreType.DMA,
        pltpu.SemaphoreType.DMA,
    ],
)
def _sc_gather(x_hbm, table_hbm, out_hbm, idx_v, rows0, rows1,
               gs0, gs1, ws0, ws1):
    wid = lax.axis_index("s") * NC + lax.axis_index("c")
    base = wid * B_PER_W
    rows = (rows0, rows1)
    gs = (gs0, gs1)
    ws = (ws0, ws1)

    pltpu.sync_copy(x_hbm.at[pl.ds(wid * XROWS_W, XROWS_W), :], idx_v)

    def for_group(g, b, fn):
        for q in range(GR):
            fn(pltpu.make_async_copy(
                table_hbm.at[idx_v.at[g * GR + q]],
                rows[b].at[pl.ds(q * HIST, HIST), :], gs[b]))

    def start_group(g, b):
        for_group(g, b, lambda cp: cp.start())

    def wait_group(g, b):
        for_group(g, b, lambda cp: cp.wait())

    start_group(0, 0)
    start_group(1, 1)

    def outer(j, carry):
        for b in range(2):
            g = 2 * j + b
            out_slc = out_hbm.at[pl.ds(base + g * GROUP, GROUP), :]
            wait_group(g, b)
            pltpu.async_copy(rows[b], out_slc, ws[b])

            @pl.when(j < NG // 2 - 1)
            def _():
                pltpu.make_async_copy(rows[b], out_slc, ws[b]).wait()
                start_group(g + 2, b)

        return carry

    lax.fori_loop(0, NG // 2, outer, 0)

    for b in range(2):
        g = NG - 2 + b
        pltpu.make_async_copy(
            rows[b], out_hbm.at[pl.ds(base + g * GROUP, GROUP), :],
            ws[b]).wait()


def kernel(x, weight):
    out = _sc_gather(x, weight)
    return out.reshape(x.shape + (weight.shape[1],))


# speedup vs baseline: 1.3316x; 1.3316x over previous
import functools

import jax
import jax.numpy as jnp
from jax import lax
from jax.experimental import pallas as pl
from jax.experimental.pallas import tpu as pltpu
from jax.experimental.pallas import tpu_sc as plsc

D = 64
BATCH = 4096
HIST = 200
B = BATCH * HIST
NC, NS = 2, 16
NW = NC * NS
XROWS_W = BATCH // NW
B_PER_W = XROWS_W * HIST
GR = 4
GROUP = GR * HIST
NG = XROWS_W // GR

_mesh = plsc.VectorSubcoreMesh(core_axis_name="c", subcore_axis_name="s")


@functools.partial(
    pl.kernel,
    mesh=_mesh,
    out_type=jax.ShapeDtypeStruct((B, 2 * D), jnp.float32),
    compiler_params=pltpu.CompilerParams(use_tc_tiling_on_sc=False),
    scratch_types=[
        pltpu.VMEM((XROWS_W, HIST), jnp.int32),
        pltpu.VMEM((GROUP, D), jnp.float32),
        pltpu.VMEM((GROUP, D), jnp.float32),
        pltpu.SemaphoreType.DMA,
        pltpu.SemaphoreType.DMA,
        pltpu.SemaphoreType.DMA,
        pltpu.SemaphoreType.DMA,
    ],
)
def _sc_gather(x_hbm, table_hbm, out_hbm, idx_v, rows0, rows1,
               gs0, gs1, ws0, ws1):
    wid = lax.axis_index("s") * NC + lax.axis_index("c")
    base = wid * B_PER_W
    rows = (rows0, rows1)
    gs = (gs0, gs1)
    ws = (ws0, ws1)

    pltpu.sync_copy(x_hbm.at[pl.ds(wid * XROWS_W, XROWS_W), :], idx_v)

    def for_group(g, b, fn):
        for q in range(GR):
            fn(pltpu.make_async_copy(
                table_hbm.at[idx_v.at[g * GR + q]],
                rows[b].at[pl.ds(q * HIST, HIST), :], gs[b]))

    def start_group(g, b):
        for_group(g, b, lambda cp: cp.start())

    def wait_group(g, b):
        for_group(g, b, lambda cp: cp.wait())

    start_group(0, 0)
    start_group(1, 1)

    def outer(j, carry):
        for b in range(2):
            g = 2 * j + b
            out_slc = out_hbm.at[pl.ds(base + g * GROUP, GROUP), pl.ds(0, D)]
            wait_group(g, b)
            pltpu.async_copy(rows[b], out_slc, ws[b])

            @pl.when(j < NG // 2 - 1)
            def _():
                pltpu.make_async_copy(rows[b], out_slc, ws[b]).wait()
                start_group(g + 2, b)

        return carry

    lax.fori_loop(0, NG // 2, outer, 0)

    for b in range(2):
        g = NG - 2 + b
        pltpu.make_async_copy(
            rows[b],
            out_hbm.at[pl.ds(base + g * GROUP, GROUP), pl.ds(0, D)],
            ws[b]).wait()


def kernel(x, weight):
    out = _sc_gather(x, weight)
    return out[:, :D].reshape(BATCH, HIST, D)
